# Initial kernel scaffold; baseline (speedup 1.0000x reference)
import jax
import jax.numpy as jnp
from jax.experimental import pallas as pl


def kernel(x, edge_index, edge_attr, pos, t, topo_cond, stab_cond, sust_cond, batch, params):
    raise NotImplementedError("write your pallas kernel here")


# speedup vs baseline: 1.7389x; 1.7389x over previous
import functools
import math

import jax
import jax.numpy as jnp
import numpy as np
from jax import lax
from jax.experimental import pallas as pl
from jax.experimental.pallas import tpu as pltpu
from jax.experimental.pallas import tpu_sc as plsc

N_NODES = 10000
N_EDGES = 160000
HIDDEN = 64
NUM_LAYERS = 6
HEADS = 4
DIM_HEAD = 32
INNER = HEADS * DIM_HEAD
N_GRAPHS = 8
TIME_DIM = 64

_NW = 32
_EPW = N_EDGES // _NW
_CHUNK = 64
_NCHUNK = 79
_EPWP = _NCHUNK * _CHUNK
_STRIPE = N_NODES // 16


def _sc_edge_call(tbl, ep3, src3d, dst3d):
    mesh = plsc.VectorSubcoreMesh(core_axis_name="c", subcore_axis_name="s")

    @functools.partial(
        pl.kernel, mesh=mesh,
        out_type=jax.ShapeDtypeStruct((2, 16, _STRIPE, 2 * HIDDEN), jnp.float32),
        scratch_types=[
            pltpu.VMEM((1, _CHUNK), jnp.int32),
            pltpu.VMEM((1, _CHUNK), jnp.int32),
            pltpu.VMEM((_CHUNK, 2 * HIDDEN), jnp.float32),
            pltpu.VMEM((_CHUNK, 2 * HIDDEN), jnp.float32),
            pltpu.VMEM((_CHUNK, HIDDEN), jnp.float32),
            pltpu.VMEM_SHARED((N_NODES, 2 * HIDDEN), jnp.float32),
            pltpu.SemaphoreType.DMA,
            pltpu.SemaphoreType.DMA,
            pltpu.SemaphoreType.DMA,
        ],
    )
    def k(tbl_hbm, ep_hbm, src_hbm, dst_hbm, out_hbm,
          src_row, dst_row, rows_s, rows_d, rows_e, s_sh,
          sem1, sem2, sem3):
        c = lax.axis_index("c")
        s = lax.axis_index("s")
        w = c * 16 + s

        def fill(r, carry):
            for kk in range(2 * HIDDEN // 16):
                rows_s[r, pl.ds(16 * kk, 16)] = jnp.zeros((16,), jnp.float32)
            return carry
        lax.fori_loop(0, _CHUNK, fill, 0)
        base = s * _STRIPE
        for off in range(0, 576, _CHUNK):
            pltpu.sync_copy(rows_s, s_sh.at[pl.ds(base + off, _CHUNK)])
        pltpu.sync_copy(rows_s.at[pl.ds(0, 49)], s_sh.at[pl.ds(base + 576, 49)])
        plsc.subcore_barrier()

        def chunk(i, carry):
            pltpu.sync_copy(src_hbm.at[w, i], src_row)
            pltpu.sync_copy(dst_hbm.at[w, i], dst_row)
            cp1 = pltpu.async_copy(tbl_hbm.at[src_row.at[0]], rows_s, sem1)
            cp2 = pltpu.async_copy(tbl_hbm.at[dst_row.at[0]], rows_d, sem2)
            cp3 = pltpu.async_copy(ep_hbm.at[w * _NCHUNK + i], rows_e, sem3)
            cp1.wait()
            cp2.wait()
            cp3.wait()

            def body(r, carry2):
                flag = jnp.where(i * _CHUNK + r < _EPW, 1.0, 0.0)
                for kk in range(HIDDEN // 16):
                    sl = pl.ds(16 * kk, 16)
                    z = (rows_s[r, sl] + rows_d[r, pl.ds(HIDDEN + 16 * kk, 16)]
                         + rows_e[r, sl])
                    rows_s[r, sl] = flag * (z / (1.0 + jnp.exp(-z)))
                    rows_s[r, pl.ds(HIDDEN + 16 * kk, 16)] = jnp.broadcast_to(
                        flag, (16,))
                return carry2
            lax.fori_loop(0, _CHUNK, body, 0)
            pltpu.sync_copy(rows_s, s_sh.at[dst_row.at[0]], add=True)
            return carry
        lax.fori_loop(0, _NCHUNK, chunk, 0)

        plsc.subcore_barrier()
        pltpu.sync_copy(s_sh.at[pl.ds(base, _STRIPE)], out_hbm.at[c, s])

    return k(tbl, ep3, src3d, dst3d).reshape(2, N_NODES, 2 * HIDDEN)


def _cond_body(t_ref, topo_ref, stab_ref, sust_ref, freqs_ref,
               tl1w, tl1b, tl2w, tl2b,
               pl1w, pl1b, pl2w, pl2b,
               sl1w, sl1b, sl2w, sl2b,
               ul1w, ul1b, ul2w, ul2b,
               cl1w, cl1b, cl2w, cl2b,
               akw, avw,
               silu_te_ref, k8_ref, v8_ref):
    silu = jax.nn.silu
    te = t_ref[...] * freqs_ref[...]
    te = jnp.concatenate([jnp.sin(te), jnp.cos(te)], axis=-1)
    te = silu(jnp.dot(te, tl1w[...], preferred_element_type=jnp.float32) + tl1b[...])
    te = jnp.dot(te, tl2w[...], preferred_element_type=jnp.float32) + tl2b[...]
    silu_te_ref[...] = silu(te)
    topo = silu(jnp.dot(topo_ref[...], pl1w[...], preferred_element_type=jnp.float32) + pl1b[...])
    topo = jnp.dot(topo, pl2w[...], preferred_element_type=jnp.float32) + pl2b[...]
    stab = silu(jnp.dot(stab_ref[...], sl1w[...], preferred_element_type=jnp.float32) + sl1b[...])
    stab = jnp.dot(stab, sl2w[...], preferred_element_type=jnp.float32) + sl2b[...]
    sust = silu(jnp.dot(sust_ref[...], ul1w[...], preferred_element_type=jnp.float32) + ul1b[...])
    sust = jnp.dot(sust, ul2w[...], preferred_element_type=jnp.float32) + ul2b[...]
    cc = jnp.concatenate([topo, stab, sust], axis=-1)
    cc = silu(jnp.dot(cc, cl1w[...], preferred_element_type=jnp.float32) + cl1b[...])
    cond = jnp.dot(cc, cl2w[...], preferred_element_type=jnp.float32) + cl2b[...]
    k8_ref[...] = jnp.dot(cond, akw[...], preferred_element_type=jnp.float32)
    v8_ref[...] = jnp.dot(cond, avw[...], preferred_element_type=jnp.float32)


def _cond_call(t2d, topo, stab, sust, freqs, p):
    outs = [
        jax.ShapeDtypeStruct((N_GRAPHS, HIDDEN), jnp.float32),
        jax.ShapeDtypeStruct((N_GRAPHS, INNER), jnp.float32),
        jax.ShapeDtypeStruct((N_GRAPHS, INNER), jnp.float32),
    ]
    args = [t2d, topo, stab, sust, freqs]
    for name in ("time_l1", "time_l2", "topo_l1", "topo_l2", "stab_l1",
                 "stab_l2", "sust_l1", "sust_l2", "comb_l1", "comb_l2"):
        args.append(p[name]["w"])
        args.append(p[name]["b"][None, :])
    args.append(p["attn_k"])
    args.append(p["attn_v"])
    return pl.pallas_call(_cond_body, out_shape=outs)(*args)


def _edge_const_body(ea_ref, wc_ref, bc_ref, *outs):
    r = jnp.dot(ea_ref[...], wc_ref[...], preferred_element_type=jnp.float32) + bc_ref[...]
    for l in range(NUM_LAYERS):
        outs[l][...] = r[:, 64 * l:64 * (l + 1)]


def _edge_const_call(edge_attr, wc_all, bc_all):
    B = 2528
    n = edge_attr.shape[0]
    G = n // B
    return pl.pallas_call(
        _edge_const_body,
        grid=(G,),
        in_specs=[
            pl.BlockSpec((B, 20), lambda i: (i, 0)),
            pl.BlockSpec((20, 64 * NUM_LAYERS), lambda i: (0, 0)),
            pl.BlockSpec((1, 64 * NUM_LAYERS), lambda i: (0, 0)),
        ],
        out_specs=[pl.BlockSpec((B, HIDDEN), lambda i: (i, 0))] * NUM_LAYERS,
        out_shape=[jax.ShapeDtypeStruct((n, HIDDEN), jnp.float32)] * NUM_LAYERS,
    )(edge_attr, wc_all, bc_all)


def _init_body(x_ref, b_ref, te_ref, wemb_ref, bemb_ref, wsd_ref,
               h_ref, ti_ref, hsd_ref):
    h = jnp.dot(x_ref[...], wemb_ref[...], preferred_element_type=jnp.float32) + bemb_ref[...]
    h_ref[...] = h
    oh = (b_ref[...] == lax.broadcasted_iota(jnp.int32, (1, N_GRAPHS), 1)
          ).astype(jnp.float32)
    ti_ref[...] = jnp.dot(oh, te_ref[...], preferred_element_type=jnp.float32)
    hsd_ref[...] = jnp.dot(h, wsd_ref[...], preferred_element_type=jnp.float32)


def _init_call(x, batch2d, silu_te, wemb, bemb, wsd0):
    B = 1000
    G = N_NODES // B
    return pl.pallas_call(
        _init_body,
        grid=(G,),
        in_specs=[
            pl.BlockSpec((B, 12), lambda i: (i, 0)),
            pl.BlockSpec((B, 1), lambda i: (i, 0)),
            pl.BlockSpec((N_GRAPHS, HIDDEN), lambda i: (0, 0)),
            pl.BlockSpec((12, HIDDEN), lambda i: (0, 0)),
            pl.BlockSpec((1, HIDDEN), lambda i: (0, 0)),
            pl.BlockSpec((HIDDEN, 2 * HIDDEN), lambda i: (0, 0)),
        ],
        out_specs=[pl.BlockSpec((B, HIDDEN), lambda i: (i, 0)),
                   pl.BlockSpec((B, HIDDEN), lambda i: (i, 0)),
                   pl.BlockSpec((B, 2 * HIDDEN), lambda i: (i, 0))],
        out_shape=[jax.ShapeDtypeStruct((N_NODES, HIDDEN), jnp.float32),
                   jax.ShapeDtypeStruct((N_NODES, HIDDEN), jnp.float32),
                   jax.ShapeDtypeStruct((N_NODES, 2 * HIDDEN), jnp.float32)],
    )(x, batch2d, silu_te, wemb, bemb, wsd0)


def _attention(hu, kbd_ref, vbd_ref):
    q = jnp.dot(hu, kbd_ref[...], preferred_element_type=jnp.float32)
    ps = []
    for hh in range(HEADS):
        sl = q[:, 8 * hh:8 * (hh + 1)]
        m = jnp.max(sl, axis=-1, keepdims=True)
        e = jnp.exp(sl - m)
        ps.append(e / jnp.sum(e, axis=-1, keepdims=True))
    P = jnp.concatenate(ps, axis=-1)
    return jnp.dot(P, vbd_ref[...], preferred_element_type=jnp.float32)


def _node_update(h_ref, s2_ref, ti_ref, wh_ref, wagg_ref, bvec_ref,
                 wcnt_ref, g_ref, b_ref, wqk_ref, vbd_ref, wao_ref, bao_ref):
    h = h_ref[...]
    acc = s2_ref[0] + s2_ref[1]
    S = acc[:, 0:HIDDEN]
    cnt = acc[:, HIDDEN:HIDDEN + 1]
    hu = (jnp.dot(h, wh_ref[...], preferred_element_type=jnp.float32)
          + jnp.dot(S, wagg_ref[...], preferred_element_type=jnp.float32)
          + cnt * wcnt_ref[...] + bvec_ref[...])
    mu = jnp.mean(hu, axis=-1, keepdims=True)
    var = jnp.mean((hu - mu) * (hu - mu), axis=-1, keepdims=True)
    hu = (hu - mu) * lax.rsqrt(var + 1e-5) * g_ref[...] + b_ref[...]
    att = _attention(hu, wqk_ref, vbd_ref)
    lin = jnp.dot(att, wao_ref[...], preferred_element_type=jnp.float32) + bao_ref[...]
    return h + lin + ti_ref[...]


def _node_body_proj(h_ref, s2_ref, ti_ref, wh_ref, wagg_ref,
                    bvec_ref, wcnt_ref, g_ref, b_ref, wqk_ref, vbd_ref,
                    wao_ref, bao_ref, wsd_ref,
                    hn_ref, hsd_ref):
    hn = _node_update(h_ref, s2_ref, ti_ref, wh_ref, wagg_ref,
                      bvec_ref, wcnt_ref, g_ref, b_ref, wqk_ref, vbd_ref,
                      wao_ref, bao_ref)
    hn_ref[...] = hn
    hsd_ref[...] = jnp.dot(hn, wsd_ref[...], preferred_element_type=jnp.float32)


def _node_body_last(h_ref, s2_ref, ti_ref, wh_ref, wagg_ref,
                    bvec_ref, wcnt_ref, g_ref, b_ref, wqk_ref, vbd_ref,
                    wao_ref, bao_ref, hn_ref):
    hn_ref[...] = _node_update(h_ref, s2_ref, ti_ref, wh_ref,
                               wagg_ref, bvec_ref, wcnt_ref, g_ref, b_ref,
                               wqk_ref, vbd_ref, wao_ref, bao_ref)


def _node_call(h, s2, ti, lw, proj):
    B = 1000
    G = N_NODES // B
    common_in = [
        pl.BlockSpec((B, HIDDEN), lambda i: (i, 0)),
        pl.BlockSpec((2, B, 2 * HIDDEN), lambda i: (0, i, 0)),
        pl.BlockSpec((B, HIDDEN), lambda i: (i, 0)),
        pl.BlockSpec((HIDDEN, HIDDEN), lambda i: (0, 0)),
        pl.BlockSpec((HIDDEN, HIDDEN), lambda i: (0, 0)),
        pl.BlockSpec((1, HIDDEN), lambda i: (0, 0)),
        pl.BlockSpec((1, HIDDEN), lambda i: (0, 0)),
        pl.BlockSpec((1, HIDDEN), lambda i: (0, 0)),
        pl.BlockSpec((1, HIDDEN), lambda i: (0, 0)),
        pl.BlockSpec((HIDDEN, 32), lambda i: (0, 0)),
        pl.BlockSpec((32, INNER), lambda i: (0, 0)),
        pl.BlockSpec((INNER, HIDDEN), lambda i: (0, 0)),
        pl.BlockSpec((1, HIDDEN), lambda i: (0, 0)),
    ]
    args = [h, s2, ti, lw["wh"], lw["wagg"], lw["bvec"], lw["wcnt"],
            lw["g"], lw["b"], lw["wqk"], lw["vbd"], lw["wao"], lw["bao"]]
    if proj:
        in_specs = common_in + [
            pl.BlockSpec((HIDDEN, 2 * HIDDEN), lambda i: (0, 0)),
        ]
        return pl.pallas_call(
            _node_body_proj,
            grid=(G,),
            in_specs=in_specs,
            out_specs=[pl.BlockSpec((B, HIDDEN), lambda i: (i, 0)),
                       pl.BlockSpec((B, 2 * HIDDEN), lambda i: (i, 0))],
            out_shape=[jax.ShapeDtypeStruct((N_NODES, HIDDEN), jnp.float32),
                       jax.ShapeDtypeStruct((N_NODES, 2 * HIDDEN), jnp.float32)],
        )(*args, lw["wsd"])
    return pl.pallas_call(
        _node_body_last,
        grid=(G,),
        in_specs=common_in,
        out_specs=pl.BlockSpec((B, HIDDEN), lambda i: (i, 0)),
        out_shape=jax.ShapeDtypeStruct((N_NODES, HIDDEN), jnp.float32),
    )(*args)


def _heads_body(h_ref, n1w, n1b, n2w, n2b, p1w, p1b, p2w, p2b,
                np_ref, pp_ref):
    silu = jax.nn.silu
    h = h_ref[...]
    a = silu(jnp.dot(h, n1w[...], preferred_element_type=jnp.float32) + n1b[...])
    np_ref[...] = jnp.dot(a, n2w[...], preferred_element_type=jnp.float32) + n2b[...]
    b = silu(jnp.dot(h, p1w[...], preferred_element_type=jnp.float32) + p1b[...])
    pp_ref[...] = jnp.dot(b, p2w[...], preferred_element_type=jnp.float32) + p2b[...]


def _heads_call(h, p):
    B = 1000
    G = N_NODES // B
    return pl.pallas_call(
        _heads_body,
        grid=(G,),
        in_specs=[
            pl.BlockSpec((B, HIDDEN), lambda i: (i, 0)),
            pl.BlockSpec((HIDDEN, 2 * HIDDEN), lambda i: (0, 0)),
            pl.BlockSpec((1, 2 * HIDDEN), lambda i: (0, 0)),
            pl.BlockSpec((2 * HIDDEN, 12), lambda i: (0, 0)),
            pl.BlockSpec((1, 12), lambda i: (0, 0)),
            pl.BlockSpec((HIDDEN, HIDDEN), lambda i: (0, 0)),
            pl.BlockSpec((1, HIDDEN), lambda i: (0, 0)),
            pl.BlockSpec((HIDDEN, 3), lambda i: (0, 0)),
            pl.BlockSpec((1, 3), lambda i: (0, 0)),
        ],
        out_specs=[pl.BlockSpec((B, 12), lambda i: (i, 0)),
                   pl.BlockSpec((B, 3), lambda i: (i, 0))],
        out_shape=[jax.ShapeDtypeStruct((N_NODES, 12), jnp.float32),
                   jax.ShapeDtypeStruct((N_NODES, 3), jnp.float32)],
    )(h, p["node_pred1"]["w"], p["node_pred1"]["b"][None, :],
      p["node_pred2"]["w"], p["node_pred2"]["b"][None, :],
      p["pos_pred1"]["w"], p["pos_pred1"]["b"][None, :],
      p["pos_pred2"]["w"], p["pos_pred2"]["b"][None, :])


def kernel(x, edge_index, edge_attr, pos, t, topo_cond, stab_cond, sust_cond,
           batch, params):
    p = params
    scale = DIM_HEAD ** (-0.5)
    half = TIME_DIM // 2
    freqs = jnp.asarray(
        np.exp(np.arange(half, dtype=np.float32) * -(math.log(10000.0) / (half - 1)))
    )[None, :]

    wq_scaled = p["attn_q"] * scale
    wc_blocks, bc_blocks = [], []
    lws = []
    for l in range(NUM_LAYERS):
        gp = p["gnn"][l]
        w1 = gp["msg1"]["w"]
        ws_l, wd_l, we_l = w1[0:64], w1[64:128], w1[128:192]
        wc_blocks.append(p["edge_emb"]["w"] @ we_l)
        bc_blocks.append((p["edge_emb"]["b"] @ we_l + gp["msg1"]["b"])[None, :])
        wu = gp["upd"]["w"]
        lws.append({
            "wsd": jnp.concatenate([ws_l, wd_l], axis=1),
            "wh": wu[0:64],
            "wagg": gp["msg2"]["w"] @ wu[64:128],
            "wcnt": (gp["msg2"]["b"] @ wu[64:128])[None, :],
            "bvec": gp["upd"]["b"][None, :],
            "g": p["ln"][l]["g"][None, :],
            "b": p["ln"][l]["b"][None, :],
        })
    wc_all = jnp.concatenate(wc_blocks, axis=1)
    bc_all = jnp.concatenate(bc_blocks, axis=1)

    silu_te, k8, v8 = _cond_call(t[:, None], topo_cond, stab_cond, sust_cond,
                                 freqs, p)

    kbd = jnp.zeros((INNER, 32), jnp.float32)
    vbd = jnp.zeros((32, INNER), jnp.float32)
    for hh in range(HEADS):
        kbd = kbd.at[32 * hh:32 * (hh + 1), 8 * hh:8 * (hh + 1)].set(
            k8[:, 32 * hh:32 * (hh + 1)].T)
        vbd = vbd.at[8 * hh:8 * (hh + 1), 32 * hh:32 * (hh + 1)].set(
            v8[:, 32 * hh:32 * (hh + 1)])
    wqk = wq_scaled @ kbd
    for l in range(NUM_LAYERS):
        lws[l]["wqk"] = wqk
        lws[l]["vbd"] = vbd
        lws[l]["wao"] = p["attn_out"]["w"]
        lws[l]["bao"] = p["attn_out"]["b"][None, :]

    ea_pad = jnp.pad(edge_attr.reshape(_NW, _EPW, 20),
                     ((0, 0), (0, _EPWP - _EPW), (0, 0))
                     ).reshape(_NW * _EPWP, 20)
    eps = _edge_const_call(ea_pad, wc_all, bc_all)

    h, ti, hsd = _init_call(
        x, batch[:, None], silu_te,
        p["node_emb"]["w"], p["node_emb"]["b"][None, :], lws[0]["wsd"])

    src3d = jnp.pad(edge_index[0].reshape(_NW, _EPW),
                    ((0, 0), (0, _EPWP - _EPW))
                    ).reshape(_NW, _NCHUNK, 1, _CHUNK)
    dst3d = jnp.pad(edge_index[1].reshape(_NW, _EPW),
                    ((0, 0), (0, _EPWP - _EPW))
                    ).reshape(_NW, _NCHUNK, 1, _CHUNK)

    for l in range(NUM_LAYERS):
        ep3 = eps[l].reshape(_NW * _NCHUNK, _CHUNK, HIDDEN)
        s2 = _sc_edge_call(hsd, ep3, src3d, dst3d)
        if l < NUM_LAYERS - 1:
            lw = dict(lws[l])
            lw["wsd"] = lws[l + 1]["wsd"]
            h, hsd = _node_call(h, s2, ti, lw, proj=True)
        else:
            h = _node_call(h, s2, ti, lws[l], proj=False)

    node_pred, pos_pred = _heads_call(h, p)
    return node_pred, pos_pred


# speedup vs baseline: 3.1211x; 1.7948x over previous
import functools
import math

import jax
import jax.numpy as jnp
import numpy as np
from jax import lax
from jax.experimental import pallas as pl
from jax.experimental.pallas import tpu as pltpu
from jax.experimental.pallas import tpu_sc as plsc

N_NODES = 10000
N_EDGES = 160000
HIDDEN = 64
NUM_LAYERS = 6
HEADS = 4
DIM_HEAD = 32
INNER = HEADS * DIM_HEAD
N_GRAPHS = 8
TIME_DIM = 64

_NW = 32
_EPW = N_EDGES // _NW
_CHUNK = 40
_NCHUNK = _EPW // _CHUNK
_STRIPE = N_NODES // 16


def _sc_edge_call(tbl, ep3, src3d, dst3d):
    mesh = plsc.VectorSubcoreMesh(core_axis_name="c", subcore_axis_name="s")

    @functools.partial(
        pl.kernel, mesh=mesh,
        out_type=jax.ShapeDtypeStruct((2, 16, _STRIPE, 2 * HIDDEN), jnp.float32),
        scratch_types=[
            pltpu.VMEM((2, 1, _CHUNK), jnp.int32),
            pltpu.VMEM((2, 1, _CHUNK), jnp.int32),
            pltpu.VMEM((2, _CHUNK, 2 * HIDDEN), jnp.float32),
            pltpu.VMEM((2, _CHUNK, 2 * HIDDEN), jnp.float32),
            pltpu.VMEM((2, _CHUNK, HIDDEN), jnp.float32),
            pltpu.VMEM_SHARED((N_NODES, 2 * HIDDEN), jnp.float32),
            pltpu.SemaphoreType.DMA,
            pltpu.SemaphoreType.DMA,
            pltpu.SemaphoreType.DMA,
            pltpu.SemaphoreType.DMA,
            pltpu.SemaphoreType.DMA,
            pltpu.SemaphoreType.DMA,
        ],
    )
    def k(tbl_hbm, ep_hbm, src_hbm, dst_hbm, out_hbm,
          ixs, ixd, rows_s, rows_d, rows_e, s_sh,
          six0, six1, sg0, sg1, ssc0, ssc1):
        c = lax.axis_index("c")
        s = lax.axis_index("s")
        w = c * 16 + s
        ep0 = w * _NCHUNK

        def issue_idx(i, b):
            sem = [six0, six1][b]
            return (pltpu.async_copy(src_hbm.at[w, i], ixs.at[b], sem),
                    pltpu.async_copy(dst_hbm.at[w, i], ixd.at[b], sem))

        def issue_gathers(i, b):
            sem = [sg0, sg1][b]
            return (pltpu.async_copy(tbl_hbm.at[ixs.at[b, 0]],
                                     rows_s.at[b], sem),
                    pltpu.async_copy(tbl_hbm.at[ixd.at[b, 0]],
                                     rows_d.at[b], sem),
                    pltpu.async_copy(ep_hbm.at[ep0 + i], rows_e.at[b], sem))

        def compute(b):
            def body(r, carry2):
                zs = []
                for kk in range(HIDDEN // 16):
                    zs.append(rows_s[b, r, pl.ds(16 * kk, 16)]
                              + rows_d[b, r, pl.ds(HIDDEN + 16 * kk, 16)]
                              + rows_e[b, r, pl.ds(16 * kk, 16)])
                for kk in range(HIDDEN // 16):
                    z = zs[kk]
                    rows_d[b, r, pl.ds(16 * kk, 16)] = z / (1.0 + jnp.exp(-z))
                    rows_d[b, r, pl.ds(HIDDEN + 16 * kk, 16)] = jnp.ones(
                        (16,), jnp.float32)
                return carry2
            lax.fori_loop(0, _CHUNK, body, 0)

        def issue_scatter(b):
            return pltpu.async_copy(rows_d.at[b], s_sh.at[ixd.at[b, 0]],
                                    [ssc0, ssc1][b], add=True)

        def fill(r, carry):
            for kk in range(2 * HIDDEN // 16):
                rows_s[0, r, pl.ds(16 * kk, 16)] = jnp.zeros((16,), jnp.float32)
            return carry
        lax.fori_loop(0, _CHUNK, fill, 0)
        base = s * _STRIPE
        for off in range(0, 600, _CHUNK):
            pltpu.sync_copy(rows_s.at[0], s_sh.at[pl.ds(base + off, _CHUNK)])
        pltpu.sync_copy(rows_s.at[0].at[pl.ds(0, 25)],
                        s_sh.at[pl.ds(base + 600, 25)])
        plsc.subcore_barrier()

        def pair(j, carry):
            a = 2 * j
            ia1, ia2 = issue_idx(a, 0)
            ib1, ib2 = issue_idx(a + 1, 1)
            ia1.wait()
            ia2.wait()
            ga = issue_gathers(a, 0)
            ib1.wait()
            ib2.wait()
            gb = issue_gathers(a + 1, 1)
            for h in ga:
                h.wait()
            compute(0)
            sa = issue_scatter(0)
            for h in gb:
                h.wait()
            compute(1)
            sa.wait()
            sb = issue_scatter(1)
            sb.wait()
            return carry
        lax.fori_loop(0, _NCHUNK // 2, pair, 0)

        ia1, ia2 = issue_idx(_NCHUNK - 1, 0)
        ia1.wait()
        ia2.wait()
        ga = issue_gathers(_NCHUNK - 1, 0)
        for h in ga:
            h.wait()
        compute(0)
        issue_scatter(0).wait()

        plsc.subcore_barrier()
        pltpu.sync_copy(s_sh.at[pl.ds(base, _STRIPE)], out_hbm.at[c, s])

    return k(tbl, ep3, src3d, dst3d).reshape(2, N_NODES, 2 * HIDDEN)


def _cond_body(t_ref, topo_ref, stab_ref, sust_ref, freqs_ref,
               tl1w, tl1b, tl2w, tl2b,
               pl1w, pl1b, pl2w, pl2b,
               sl1w, sl1b, sl2w, sl2b,
               ul1w, ul1b, ul2w, ul2b,
               cl1w, cl1b, cl2w, cl2b,
               akw, avw,
               silu_te_ref, k8_ref, v8_ref):
    silu = jax.nn.silu
    te = t_ref[...] * freqs_ref[...]
    te = jnp.concatenate([jnp.sin(te), jnp.cos(te)], axis=-1)
    te = silu(jnp.dot(te, tl1w[...], preferred_element_type=jnp.float32) + tl1b[...])
    te = jnp.dot(te, tl2w[...], preferred_element_type=jnp.float32) + tl2b[...]
    silu_te_ref[...] = silu(te)
    topo = silu(jnp.dot(topo_ref[...], pl1w[...], preferred_element_type=jnp.float32) + pl1b[...])
    topo = jnp.dot(topo, pl2w[...], preferred_element_type=jnp.float32) + pl2b[...]
    stab = silu(jnp.dot(stab_ref[...], sl1w[...], preferred_element_type=jnp.float32) + sl1b[...])
    stab = jnp.dot(stab, sl2w[...], preferred_element_type=jnp.float32) + sl2b[...]
    sust = silu(jnp.dot(sust_ref[...], ul1w[...], preferred_element_type=jnp.float32) + ul1b[...])
    sust = jnp.dot(sust, ul2w[...], preferred_element_type=jnp.float32) + ul2b[...]
    cc = jnp.concatenate([topo, stab, sust], axis=-1)
    cc = silu(jnp.dot(cc, cl1w[...], preferred_element_type=jnp.float32) + cl1b[...])
    cond = jnp.dot(cc, cl2w[...], preferred_element_type=jnp.float32) + cl2b[...]
    k8_ref[...] = jnp.dot(cond, akw[...], preferred_element_type=jnp.float32)
    v8_ref[...] = jnp.dot(cond, avw[...], preferred_element_type=jnp.float32)


def _cond_call(t2d, topo, stab, sust, freqs, p):
    outs = [
        jax.ShapeDtypeStruct((N_GRAPHS, HIDDEN), jnp.float32),
        jax.ShapeDtypeStruct((N_GRAPHS, INNER), jnp.float32),
        jax.ShapeDtypeStruct((N_GRAPHS, INNER), jnp.float32),
    ]
    args = [t2d, topo, stab, sust, freqs]
    for name in ("time_l1", "time_l2", "topo_l1", "topo_l2", "stab_l1",
                 "stab_l2", "sust_l1", "sust_l2", "comb_l1", "comb_l2"):
        args.append(p[name]["w"])
        args.append(p[name]["b"][None, :])
    args.append(p["attn_k"])
    args.append(p["attn_v"])
    return pl.pallas_call(_cond_body, out_shape=outs)(*args)


def _edge_const_body(ea_ref, wc_ref, bc_ref, *outs):
    r = jnp.dot(ea_ref[...], wc_ref[...], preferred_element_type=jnp.float32) + bc_ref[...]
    for l in range(NUM_LAYERS):
        outs[l][...] = r[:, 64 * l:64 * (l + 1)]


def _edge_const_call(edge_attr, wc_all, bc_all):
    B = 2000
    n = edge_attr.shape[0]
    G = n // B
    return pl.pallas_call(
        _edge_const_body,
        grid=(G,),
        in_specs=[
            pl.BlockSpec((B, 20), lambda i: (i, 0)),
            pl.BlockSpec((20, 64 * NUM_LAYERS), lambda i: (0, 0)),
            pl.BlockSpec((1, 64 * NUM_LAYERS), lambda i: (0, 0)),
        ],
        out_specs=[pl.BlockSpec((B, HIDDEN), lambda i: (i, 0))] * NUM_LAYERS,
        out_shape=[jax.ShapeDtypeStruct((n, HIDDEN), jnp.float32)] * NUM_LAYERS,
    )(edge_attr, wc_all, bc_all)


def _init_body(x_ref, b_ref, te_ref, wemb_ref, bemb_ref, wsd_ref,
               h_ref, ti_ref, hsd_ref):
    h = jnp.dot(x_ref[...], wemb_ref[...], preferred_element_type=jnp.float32) + bemb_ref[...]
    h_ref[...] = h
    oh = (b_ref[...] == lax.broadcasted_iota(jnp.int32, (1, N_GRAPHS), 1)
          ).astype(jnp.float32)
    ti_ref[...] = jnp.dot(oh, te_ref[...], preferred_element_type=jnp.float32)
    hsd_ref[...] = jnp.dot(h, wsd_ref[...], preferred_element_type=jnp.float32)


def _init_call(x, batch2d, silu_te, wemb, bemb, wsd0):
    B = 1000
    G = N_NODES // B
    return pl.pallas_call(
        _init_body,
        grid=(G,),
        in_specs=[
            pl.BlockSpec((B, 12), lambda i: (i, 0)),
            pl.BlockSpec((B, 1), lambda i: (i, 0)),
            pl.BlockSpec((N_GRAPHS, HIDDEN), lambda i: (0, 0)),
            pl.BlockSpec((12, HIDDEN), lambda i: (0, 0)),
            pl.BlockSpec((1, HIDDEN), lambda i: (0, 0)),
            pl.BlockSpec((HIDDEN, 2 * HIDDEN), lambda i: (0, 0)),
        ],
        out_specs=[pl.BlockSpec((B, HIDDEN), lambda i: (i, 0)),
                   pl.BlockSpec((B, HIDDEN), lambda i: (i, 0)),
                   pl.BlockSpec((B, 2 * HIDDEN), lambda i: (i, 0))],
        out_shape=[jax.ShapeDtypeStruct((N_NODES, HIDDEN), jnp.float32),
                   jax.ShapeDtypeStruct((N_NODES, HIDDEN), jnp.float32),
                   jax.ShapeDtypeStruct((N_NODES, 2 * HIDDEN), jnp.float32)],
    )(x, batch2d, silu_te, wemb, bemb, wsd0)


def _attention(hu, kbd_ref, vbd_ref):
    q = jnp.dot(hu, kbd_ref[...], preferred_element_type=jnp.float32)
    ps = []
    for hh in range(HEADS):
        sl = q[:, 8 * hh:8 * (hh + 1)]
        m = jnp.max(sl, axis=-1, keepdims=True)
        e = jnp.exp(sl - m)
        ps.append(e / jnp.sum(e, axis=-1, keepdims=True))
    P = jnp.concatenate(ps, axis=-1)
    return jnp.dot(P, vbd_ref[...], preferred_element_type=jnp.float32)


def _node_update(h_ref, s2_ref, ti_ref, wh_ref, wagg_ref, bvec_ref,
                 wcnt_ref, g_ref, b_ref, wqk_ref, vbd_ref, wao_ref, bao_ref):
    h = h_ref[...]
    acc = s2_ref[0] + s2_ref[1]
    S = acc[:, 0:HIDDEN]
    cnt = acc[:, HIDDEN:HIDDEN + 1]
    hu = (jnp.dot(h, wh_ref[...], preferred_element_type=jnp.float32)
          + jnp.dot(S, wagg_ref[...], preferred_element_type=jnp.float32)
          + cnt * wcnt_ref[...] + bvec_ref[...])
    mu = jnp.mean(hu, axis=-1, keepdims=True)
    var = jnp.mean((hu - mu) * (hu - mu), axis=-1, keepdims=True)
    hu = (hu - mu) * lax.rsqrt(var + 1e-5) * g_ref[...] + b_ref[...]
    att = _attention(hu, wqk_ref, vbd_ref)
    lin = jnp.dot(att, wao_ref[...], preferred_element_type=jnp.float32) + bao_ref[...]
    return h + lin + ti_ref[...]


def _node_body_proj(h_ref, s2_ref, ti_ref, wh_ref, wagg_ref,
                    bvec_ref, wcnt_ref, g_ref, b_ref, wqk_ref, vbd_ref,
                    wao_ref, bao_ref, wsd_ref,
                    hn_ref, hsd_ref):
    hn = _node_update(h_ref, s2_ref, ti_ref, wh_ref, wagg_ref,
                      bvec_ref, wcnt_ref, g_ref, b_ref, wqk_ref, vbd_ref,
                      wao_ref, bao_ref)
    hn_ref[...] = hn
    hsd_ref[...] = jnp.dot(hn, wsd_ref[...], preferred_element_type=jnp.float32)


def _node_body_last(h_ref, s2_ref, ti_ref, wh_ref, wagg_ref,
                    bvec_ref, wcnt_ref, g_ref, b_ref, wqk_ref, vbd_ref,
                    wao_ref, bao_ref, hn_ref):
    hn_ref[...] = _node_update(h_ref, s2_ref, ti_ref, wh_ref,
                               wagg_ref, bvec_ref, wcnt_ref, g_ref, b_ref,
                               wqk_ref, vbd_ref, wao_ref, bao_ref)


def _node_call(h, s2, ti, lw, proj):
    B = 1000
    G = N_NODES // B
    common_in = [
        pl.BlockSpec((B, HIDDEN), lambda i: (i, 0)),
        pl.BlockSpec((2, B, 2 * HIDDEN), lambda i: (0, i, 0)),
        pl.BlockSpec((B, HIDDEN), lambda i: (i, 0)),
        pl.BlockSpec((HIDDEN, HIDDEN), lambda i: (0, 0)),
        pl.BlockSpec((HIDDEN, HIDDEN), lambda i: (0, 0)),
        pl.BlockSpec((1, HIDDEN), lambda i: (0, 0)),
        pl.BlockSpec((1, HIDDEN), lambda i: (0, 0)),
        pl.BlockSpec((1, HIDDEN), lambda i: (0, 0)),
        pl.BlockSpec((1, HIDDEN), lambda i: (0, 0)),
        pl.BlockSpec((HIDDEN, 32), lambda i: (0, 0)),
        pl.BlockSpec((32, INNER), lambda i: (0, 0)),
        pl.BlockSpec((INNER, HIDDEN), lambda i: (0, 0)),
        pl.BlockSpec((1, HIDDEN), lambda i: (0, 0)),
    ]
    args = [h, s2, ti, lw["wh"], lw["wagg"], lw["bvec"], lw["wcnt"],
            lw["g"], lw["b"], lw["wqk"], lw["vbd"], lw["wao"], lw["bao"]]
    if proj:
        in_specs = common_in + [
            pl.BlockSpec((HIDDEN, 2 * HIDDEN), lambda i: (0, 0)),
        ]
        return pl.pallas_call(
            _node_body_proj,
            grid=(G,),
            in_specs=in_specs,
            out_specs=[pl.BlockSpec((B, HIDDEN), lambda i: (i, 0)),
                       pl.BlockSpec((B, 2 * HIDDEN), lambda i: (i, 0))],
            out_shape=[jax.ShapeDtypeStruct((N_NODES, HIDDEN), jnp.float32),
                       jax.ShapeDtypeStruct((N_NODES, 2 * HIDDEN), jnp.float32)],
        )(*args, lw["wsd"])
    return pl.pallas_call(
        _node_body_last,
        grid=(G,),
        in_specs=common_in,
        out_specs=pl.BlockSpec((B, HIDDEN), lambda i: (i, 0)),
        out_shape=jax.ShapeDtypeStruct((N_NODES, HIDDEN), jnp.float32),
    )(*args)


def _heads_body(h_ref, n1w, n1b, n2w, n2b, p1w, p1b, p2w, p2b,
                np_ref, pp_ref):
    silu = jax.nn.silu
    h = h_ref[...]
    a = silu(jnp.dot(h, n1w[...], preferred_element_type=jnp.float32) + n1b[...])
    np_ref[...] = jnp.dot(a, n2w[...], preferred_element_type=jnp.float32) + n2b[...]
    b = silu(jnp.dot(h, p1w[...], preferred_element_type=jnp.float32) + p1b[...])
    pp_ref[...] = jnp.dot(b, p2w[...], preferred_element_type=jnp.float32) + p2b[...]


def _heads_call(h, p):
    B = 1000
    G = N_NODES // B
    return pl.pallas_call(
        _heads_body,
        grid=(G,),
        in_specs=[
            pl.BlockSpec((B, HIDDEN), lambda i: (i, 0)),
            pl.BlockSpec((HIDDEN, 2 * HIDDEN), lambda i: (0, 0)),
            pl.BlockSpec((1, 2 * HIDDEN), lambda i: (0, 0)),
            pl.BlockSpec((2 * HIDDEN, 12), lambda i: (0, 0)),
            pl.BlockSpec((1, 12), lambda i: (0, 0)),
            pl.BlockSpec((HIDDEN, HIDDEN), lambda i: (0, 0)),
            pl.BlockSpec((1, HIDDEN), lambda i: (0, 0)),
            pl.BlockSpec((HIDDEN, 3), lambda i: (0, 0)),
            pl.BlockSpec((1, 3), lambda i: (0, 0)),
        ],
        out_specs=[pl.BlockSpec((B, 12), lambda i: (i, 0)),
                   pl.BlockSpec((B, 3), lambda i: (i, 0))],
        out_shape=[jax.ShapeDtypeStruct((N_NODES, 12), jnp.float32),
                   jax.ShapeDtypeStruct((N_NODES, 3), jnp.float32)],
    )(h, p["node_pred1"]["w"], p["node_pred1"]["b"][None, :],
      p["node_pred2"]["w"], p["node_pred2"]["b"][None, :],
      p["pos_pred1"]["w"], p["pos_pred1"]["b"][None, :],
      p["pos_pred2"]["w"], p["pos_pred2"]["b"][None, :])


def kernel(x, edge_index, edge_attr, pos, t, topo_cond, stab_cond, sust_cond,
           batch, params):
    p = params
    scale = DIM_HEAD ** (-0.5)
    half = TIME_DIM // 2
    freqs = jnp.asarray(
        np.exp(np.arange(half, dtype=np.float32) * -(math.log(10000.0) / (half - 1)))
    )[None, :]

    wq_scaled = p["attn_q"] * scale
    wc_blocks, bc_blocks = [], []
    lws = []
    for l in range(NUM_LAYERS):
        gp = p["gnn"][l]
        w1 = gp["msg1"]["w"]
        ws_l, wd_l, we_l = w1[0:64], w1[64:128], w1[128:192]
        wc_blocks.append(p["edge_emb"]["w"] @ we_l)
        bc_blocks.append((p["edge_emb"]["b"] @ we_l + gp["msg1"]["b"])[None, :])
        wu = gp["upd"]["w"]
        lws.append({
            "wsd": jnp.concatenate([ws_l, wd_l], axis=1),
            "wh": wu[0:64],
            "wagg": gp["msg2"]["w"] @ wu[64:128],
            "wcnt": (gp["msg2"]["b"] @ wu[64:128])[None, :],
            "bvec": gp["upd"]["b"][None, :],
            "g": p["ln"][l]["g"][None, :],
            "b": p["ln"][l]["b"][None, :],
        })
    wc_all = jnp.concatenate(wc_blocks, axis=1)
    bc_all = jnp.concatenate(bc_blocks, axis=1)

    silu_te, k8, v8 = _cond_call(t[:, None], topo_cond, stab_cond, sust_cond,
                                 freqs, p)

    kbd = jnp.zeros((INNER, 32), jnp.float32)
    vbd = jnp.zeros((32, INNER), jnp.float32)
    for hh in range(HEADS):
        kbd = kbd.at[32 * hh:32 * (hh + 1), 8 * hh:8 * (hh + 1)].set(
            k8[:, 32 * hh:32 * (hh + 1)].T)
        vbd = vbd.at[8 * hh:8 * (hh + 1), 32 * hh:32 * (hh + 1)].set(
            v8[:, 32 * hh:32 * (hh + 1)])
    wqk = wq_scaled @ kbd
    for l in range(NUM_LAYERS):
        lws[l]["wqk"] = wqk
        lws[l]["vbd"] = vbd
        lws[l]["wao"] = p["attn_out"]["w"]
        lws[l]["bao"] = p["attn_out"]["b"][None, :]

    eps = _edge_const_call(edge_attr, wc_all, bc_all)

    h, ti, hsd = _init_call(
        x, batch[:, None], silu_te,
        p["node_emb"]["w"], p["node_emb"]["b"][None, :], lws[0]["wsd"])

    src3d = edge_index[0].reshape(_NW, _NCHUNK, 1, _CHUNK)
    dst3d = edge_index[1].reshape(_NW, _NCHUNK, 1, _CHUNK)

    for l in range(NUM_LAYERS):
        ep3 = eps[l].reshape(_NW * _NCHUNK, _CHUNK, HIDDEN)
        s2 = _sc_edge_call(hsd, ep3, src3d, dst3d)
        if l < NUM_LAYERS - 1:
            lw = dict(lws[l])
            lw["wsd"] = lws[l + 1]["wsd"]
            h, hsd = _node_call(h, s2, ti, lw, proj=True)
        else:
            h = _node_call(h, s2, ti, lws[l], proj=False)

    node_pred, pos_pred = _heads_call(h, p)
    return node_pred, pos_pred


# speedup vs baseline: 3.3299x; 1.0669x over previous
import functools
import math

import jax
import jax.numpy as jnp
import numpy as np
from jax import lax
from jax.experimental import pallas as pl
from jax.experimental.pallas import tpu as pltpu
from jax.experimental.pallas import tpu_sc as plsc

N_NODES = 10000
N_EDGES = 160000
HIDDEN = 64
NUM_LAYERS = 6
HEADS = 4
DIM_HEAD = 32
INNER = HEADS * DIM_HEAD
N_GRAPHS = 8
TIME_DIM = 64

_NW = 32
_EPW = N_EDGES // _NW
_CHUNK = 40
_NCHUNK = _EPW // _CHUNK
_STRIPE = N_NODES // 16


def _sc_edge_call(tbl, ep3, src3d, dst3d):
    mesh = plsc.VectorSubcoreMesh(core_axis_name="c", subcore_axis_name="s")

    @functools.partial(
        pl.kernel, mesh=mesh,
        out_type=jax.ShapeDtypeStruct((2, 16, _STRIPE, 2 * HIDDEN), jnp.float32),
        scratch_types=[
            pltpu.VMEM((4, 1, _CHUNK), jnp.int32),
            pltpu.VMEM((4, 1, _CHUNK), jnp.int32),
            pltpu.VMEM((2, _CHUNK, 2 * HIDDEN), jnp.float32),
            pltpu.VMEM((2, _CHUNK, 2 * HIDDEN), jnp.float32),
            pltpu.VMEM((2, _CHUNK, HIDDEN), jnp.float32),
            pltpu.VMEM_SHARED((N_NODES, 2 * HIDDEN), jnp.float32),
            pltpu.SemaphoreType.DMA,
            pltpu.SemaphoreType.DMA,
            pltpu.SemaphoreType.DMA,
            pltpu.SemaphoreType.DMA,
            pltpu.SemaphoreType.DMA,
            pltpu.SemaphoreType.DMA,
        ],
    )
    def k(tbl_hbm, ep_hbm, src_hbm, dst_hbm, out_hbm,
          ixs, ixd, rows_s, rows_d, rows_e, s_sh,
          six0, six1, sg0, sg1, ssc0, ssc1):
        c = lax.axis_index("c")
        s = lax.axis_index("s")
        w = c * 16 + s
        ep0 = w * _NCHUNK

        def issue_idx(i, slot):
            sem = [six0, six1][slot % 2]
            return (pltpu.async_copy(src_hbm.at[w, i], ixs.at[slot], sem),
                    pltpu.async_copy(dst_hbm.at[w, i], ixd.at[slot], sem))

        def issue_gathers(i, b, slot):
            sem = [sg0, sg1][b]
            return (pltpu.async_copy(tbl_hbm.at[ixs.at[slot, 0]],
                                     rows_s.at[b], sem),
                    pltpu.async_copy(tbl_hbm.at[ixd.at[slot, 0]],
                                     rows_d.at[b], sem),
                    pltpu.async_copy(ep_hbm.at[ep0 + i], rows_e.at[b], sem))

        def compute(b):
            def body(r0, carry2):
                for r in (2 * r0, 2 * r0 + 1):
                    zs = []
                    for kk in range(HIDDEN // 16):
                        zs.append(rows_s[b, r, pl.ds(16 * kk, 16)]
                                  + rows_d[b, r, pl.ds(HIDDEN + 16 * kk, 16)]
                                  + rows_e[b, r, pl.ds(16 * kk, 16)])
                    for kk in range(HIDDEN // 16):
                        z = zs[kk]
                        rows_d[b, r, pl.ds(16 * kk, 16)] = z / (1.0 + jnp.exp(-z))
                        rows_d[b, r, pl.ds(HIDDEN + 16 * kk, 16)] = jnp.ones(
                            (16,), jnp.float32)
                return carry2
            lax.fori_loop(0, _CHUNK // 2, body, 0)

        def issue_scatter(b, slot):
            return pltpu.async_copy(rows_d.at[b], s_sh.at[ixd.at[slot, 0]],
                                    [ssc0, ssc1][b], add=True)

        def fill(r, carry):
            for kk in range(2 * HIDDEN // 16):
                rows_s[0, r, pl.ds(16 * kk, 16)] = jnp.zeros((16,), jnp.float32)
            return carry
        lax.fori_loop(0, _CHUNK, fill, 0)
        base = s * _STRIPE
        for off in range(0, 600, _CHUNK):
            pltpu.sync_copy(rows_s.at[0], s_sh.at[pl.ds(base + off, _CHUNK)])
        pltpu.sync_copy(rows_s.at[0].at[pl.ds(0, 25)],
                        s_sh.at[pl.ds(base + 600, 25)])
        plsc.subcore_barrier()

        def quad(j, carry):
            a = 4 * j
            i0 = issue_idx(a, 0)
            i1 = issue_idx(a + 1, 1)
            i2 = issue_idx(a + 2, 2)
            i3 = issue_idx(a + 3, 3)
            i0[0].wait()
            i0[1].wait()
            g0 = issue_gathers(a, 0, 0)
            i1[0].wait()
            i1[1].wait()
            g1 = issue_gathers(a + 1, 1, 1)
            for h in g0:
                h.wait()
            compute(0)
            s0 = issue_scatter(0, 0)
            for h in g1:
                h.wait()
            compute(1)
            s1 = issue_scatter(1, 1)
            s0.wait()
            i2[0].wait()
            i2[1].wait()
            g2 = issue_gathers(a + 2, 0, 2)
            s1.wait()
            i3[0].wait()
            i3[1].wait()
            g3 = issue_gathers(a + 3, 1, 3)
            for h in g2:
                h.wait()
            compute(0)
            s2 = issue_scatter(0, 2)
            for h in g3:
                h.wait()
            compute(1)
            s3 = issue_scatter(1, 3)
            s2.wait()
            s3.wait()
            return carry
        lax.fori_loop(0, _NCHUNK // 4, quad, 0)

        ia1, ia2 = issue_idx(_NCHUNK - 1, 0)
        ia1.wait()
        ia2.wait()
        ga = issue_gathers(_NCHUNK - 1, 0, 0)
        for h in ga:
            h.wait()
        compute(0)
        issue_scatter(0, 0).wait()

        plsc.subcore_barrier()
        pltpu.sync_copy(s_sh.at[pl.ds(base, _STRIPE)], out_hbm.at[c, s])

    return k(tbl, ep3, src3d, dst3d).reshape(2, N_NODES, 2 * HIDDEN)


def _cond_body(t_ref, topo_ref, stab_ref, sust_ref, freqs_ref,
               tl1w, tl1b, tl2w, tl2b,
               pl1w, pl1b, pl2w, pl2b,
               sl1w, sl1b, sl2w, sl2b,
               ul1w, ul1b, ul2w, ul2b,
               cl1w, cl1b, cl2w, cl2b,
               akw, avw,
               silu_te_ref, k8_ref, v8_ref):
    silu = jax.nn.silu
    te = t_ref[...] * freqs_ref[...]
    te = jnp.concatenate([jnp.sin(te), jnp.cos(te)], axis=-1)
    te = silu(jnp.dot(te, tl1w[...], preferred_element_type=jnp.float32) + tl1b[...])
    te = jnp.dot(te, tl2w[...], preferred_element_type=jnp.float32) + tl2b[...]
    silu_te_ref[...] = silu(te)
    topo = silu(jnp.dot(topo_ref[...], pl1w[...], preferred_element_type=jnp.float32) + pl1b[...])
    topo = jnp.dot(topo, pl2w[...], preferred_element_type=jnp.float32) + pl2b[...]
    stab = silu(jnp.dot(stab_ref[...], sl1w[...], preferred_element_type=jnp.float32) + sl1b[...])
    stab = jnp.dot(stab, sl2w[...], preferred_element_type=jnp.float32) + sl2b[...]
    sust = silu(jnp.dot(sust_ref[...], ul1w[...], preferred_element_type=jnp.float32) + ul1b[...])
    sust = jnp.dot(sust, ul2w[...], preferred_element_type=jnp.float32) + ul2b[...]
    cc = jnp.concatenate([topo, stab, sust], axis=-1)
    cc = silu(jnp.dot(cc, cl1w[...], preferred_element_type=jnp.float32) + cl1b[...])
    cond = jnp.dot(cc, cl2w[...], preferred_element_type=jnp.float32) + cl2b[...]
    k8_ref[...] = jnp.dot(cond, akw[...], preferred_element_type=jnp.float32)
    v8_ref[...] = jnp.dot(cond, avw[...], preferred_element_type=jnp.float32)


def _cond_call(t2d, topo, stab, sust, freqs, p):
    outs = [
        jax.ShapeDtypeStruct((N_GRAPHS, HIDDEN), jnp.float32),
        jax.ShapeDtypeStruct((N_GRAPHS, INNER), jnp.float32),
        jax.ShapeDtypeStruct((N_GRAPHS, INNER), jnp.float32),
    ]
    args = [t2d, topo, stab, sust, freqs]
    for name in ("time_l1", "time_l2", "topo_l1", "topo_l2", "stab_l1",
                 "stab_l2", "sust_l1", "sust_l2", "comb_l1", "comb_l2"):
        args.append(p[name]["w"])
        args.append(p[name]["b"][None, :])
    args.append(p["attn_k"])
    args.append(p["attn_v"])
    return pl.pallas_call(_cond_body, out_shape=outs)(*args)


def _edge_const_body(ea_ref, wc_ref, bc_ref, *outs):
    r = jnp.dot(ea_ref[...], wc_ref[...], preferred_element_type=jnp.float32) + bc_ref[...]
    for l in range(NUM_LAYERS):
        outs[l][...] = r[:, 64 * l:64 * (l + 1)]


def _edge_const_call(edge_attr, wc_all, bc_all):
    B = 2000
    n = edge_attr.shape[0]
    G = n // B
    return pl.pallas_call(
        _edge_const_body,
        grid=(G,),
        in_specs=[
            pl.BlockSpec((B, 20), lambda i: (i, 0)),
            pl.BlockSpec((20, 64 * NUM_LAYERS), lambda i: (0, 0)),
            pl.BlockSpec((1, 64 * NUM_LAYERS), lambda i: (0, 0)),
        ],
        out_specs=[pl.BlockSpec((B, HIDDEN), lambda i: (i, 0))] * NUM_LAYERS,
        out_shape=[jax.ShapeDtypeStruct((n, HIDDEN), jnp.float32)] * NUM_LAYERS,
    )(edge_attr, wc_all, bc_all)


def _init_body(x_ref, b_ref, te_ref, wemb_ref, bemb_ref, wsd_ref,
               h_ref, ti_ref, hsd_ref):
    h = jnp.dot(x_ref[...], wemb_ref[...], preferred_element_type=jnp.float32) + bemb_ref[...]
    h_ref[...] = h
    oh = (b_ref[...] == lax.broadcasted_iota(jnp.int32, (1, N_GRAPHS), 1)
          ).astype(jnp.float32)
    ti_ref[...] = jnp.dot(oh, te_ref[...], preferred_element_type=jnp.float32)
    hsd_ref[...] = jnp.dot(h, wsd_ref[...], preferred_element_type=jnp.float32)


def _init_call(x, batch2d, silu_te, wemb, bemb, wsd0):
    B = 1000
    G = N_NODES // B
    return pl.pallas_call(
        _init_body,
        grid=(G,),
        in_specs=[
            pl.BlockSpec((B, 12), lambda i: (i, 0)),
            pl.BlockSpec((B, 1), lambda i: (i, 0)),
            pl.BlockSpec((N_GRAPHS, HIDDEN), lambda i: (0, 0)),
            pl.BlockSpec((12, HIDDEN), lambda i: (0, 0)),
            pl.BlockSpec((1, HIDDEN), lambda i: (0, 0)),
            pl.BlockSpec((HIDDEN, 2 * HIDDEN), lambda i: (0, 0)),
        ],
        out_specs=[pl.BlockSpec((B, HIDDEN), lambda i: (i, 0)),
                   pl.BlockSpec((B, HIDDEN), lambda i: (i, 0)),
                   pl.BlockSpec((B, 2 * HIDDEN), lambda i: (i, 0))],
        out_shape=[jax.ShapeDtypeStruct((N_NODES, HIDDEN), jnp.float32),
                   jax.ShapeDtypeStruct((N_NODES, HIDDEN), jnp.float32),
                   jax.ShapeDtypeStruct((N_NODES, 2 * HIDDEN), jnp.float32)],
    )(x, batch2d, silu_te, wemb, bemb, wsd0)


def _attention(hu, kbd_ref, vbd_ref):
    q = jnp.dot(hu, kbd_ref[...], preferred_element_type=jnp.float32)
    ps = []
    for hh in range(HEADS):
        sl = q[:, 8 * hh:8 * (hh + 1)]
        m = jnp.max(sl, axis=-1, keepdims=True)
        e = jnp.exp(sl - m)
        ps.append(e / jnp.sum(e, axis=-1, keepdims=True))
    P = jnp.concatenate(ps, axis=-1)
    return jnp.dot(P, vbd_ref[...], preferred_element_type=jnp.float32)


def _node_update(h_ref, s2_ref, ti_ref, wh_ref, wagg_ref, bvec_ref,
                 wcnt_ref, g_ref, b_ref, wqk_ref, vbd_ref, wao_ref, bao_ref):
    h = h_ref[...]
    acc = s2_ref[0] + s2_ref[1]
    S = acc[:, 0:HIDDEN]
    cnt = acc[:, HIDDEN:HIDDEN + 1]
    hu = (jnp.dot(h, wh_ref[...], preferred_element_type=jnp.float32)
          + jnp.dot(S, wagg_ref[...], preferred_element_type=jnp.float32)
          + cnt * wcnt_ref[...] + bvec_ref[...])
    mu = jnp.mean(hu, axis=-1, keepdims=True)
    var = jnp.mean((hu - mu) * (hu - mu), axis=-1, keepdims=True)
    hu = (hu - mu) * lax.rsqrt(var + 1e-5) * g_ref[...] + b_ref[...]
    att = _attention(hu, wqk_ref, vbd_ref)
    lin = jnp.dot(att, wao_ref[...], preferred_element_type=jnp.float32) + bao_ref[...]
    return h + lin + ti_ref[...]


def _node_body_proj(h_ref, s2_ref, ti_ref, wh_ref, wagg_ref,
                    bvec_ref, wcnt_ref, g_ref, b_ref, wqk_ref, vbd_ref,
                    wao_ref, bao_ref, wsd_ref,
                    hn_ref, hsd_ref):
    hn = _node_update(h_ref, s2_ref, ti_ref, wh_ref, wagg_ref,
                      bvec_ref, wcnt_ref, g_ref, b_ref, wqk_ref, vbd_ref,
                      wao_ref, bao_ref)
    hn_ref[...] = hn
    hsd_ref[...] = jnp.dot(hn, wsd_ref[...], preferred_element_type=jnp.float32)


def _node_body_last(h_ref, s2_ref, ti_ref, wh_ref, wagg_ref,
                    bvec_ref, wcnt_ref, g_ref, b_ref, wqk_ref, vbd_ref,
                    wao_ref, bao_ref, hn_ref):
    hn_ref[...] = _node_update(h_ref, s2_ref, ti_ref, wh_ref,
                               wagg_ref, bvec_ref, wcnt_ref, g_ref, b_ref,
                               wqk_ref, vbd_ref, wao_ref, bao_ref)


def _node_call(h, s2, ti, lw, proj):
    B = 1000
    G = N_NODES // B
    common_in = [
        pl.BlockSpec((B, HIDDEN), lambda i: (i, 0)),
        pl.BlockSpec((2, B, 2 * HIDDEN), lambda i: (0, i, 0)),
        pl.BlockSpec((B, HIDDEN), lambda i: (i, 0)),
        pl.BlockSpec((HIDDEN, HIDDEN), lambda i: (0, 0)),
        pl.BlockSpec((HIDDEN, HIDDEN), lambda i: (0, 0)),
        pl.BlockSpec((1, HIDDEN), lambda i: (0, 0)),
        pl.BlockSpec((1, HIDDEN), lambda i: (0, 0)),
        pl.BlockSpec((1, HIDDEN), lambda i: (0, 0)),
        pl.BlockSpec((1, HIDDEN), lambda i: (0, 0)),
        pl.BlockSpec((HIDDEN, 32), lambda i: (0, 0)),
        pl.BlockSpec((32, INNER), lambda i: (0, 0)),
        pl.BlockSpec((INNER, HIDDEN), lambda i: (0, 0)),
        pl.BlockSpec((1, HIDDEN), lambda i: (0, 0)),
    ]
    args = [h, s2, ti, lw["wh"], lw["wagg"], lw["bvec"], lw["wcnt"],
            lw["g"], lw["b"], lw["wqk"], lw["vbd"], lw["wao"], lw["bao"]]
    if proj:
        in_specs = common_in + [
            pl.BlockSpec((HIDDEN, 2 * HIDDEN), lambda i: (0, 0)),
        ]
        return pl.pallas_call(
            _node_body_proj,
            grid=(G,),
            in_specs=in_specs,
            out_specs=[pl.BlockSpec((B, HIDDEN), lambda i: (i, 0)),
                       pl.BlockSpec((B, 2 * HIDDEN), lambda i: (i, 0))],
            out_shape=[jax.ShapeDtypeStruct((N_NODES, HIDDEN), jnp.float32),
                       jax.ShapeDtypeStruct((N_NODES, 2 * HIDDEN), jnp.float32)],
        )(*args, lw["wsd"])
    return pl.pallas_call(
        _node_body_last,
        grid=(G,),
        in_specs=common_in,
        out_specs=pl.BlockSpec((B, HIDDEN), lambda i: (i, 0)),
        out_shape=jax.ShapeDtypeStruct((N_NODES, HIDDEN), jnp.float32),
    )(*args)


def _heads_body(h_ref, n1w, n1b, n2w, n2b, p1w, p1b, p2w, p2b,
                np_ref, pp_ref):
    silu = jax.nn.silu
    h = h_ref[...]
    a = silu(jnp.dot(h, n1w[...], preferred_element_type=jnp.float32) + n1b[...])
    np_ref[...] = jnp.dot(a, n2w[...], preferred_element_type=jnp.float32) + n2b[...]
    b = silu(jnp.dot(h, p1w[...], preferred_element_type=jnp.float32) + p1b[...])
    pp_ref[...] = jnp.dot(b, p2w[...], preferred_element_type=jnp.float32) + p2b[...]


def _heads_call(h, p):
    B = 1000
    G = N_NODES // B
    return pl.pallas_call(
        _heads_body,
        grid=(G,),
        in_specs=[
            pl.BlockSpec((B, HIDDEN), lambda i: (i, 0)),
            pl.BlockSpec((HIDDEN, 2 * HIDDEN), lambda i: (0, 0)),
            pl.BlockSpec((1, 2 * HIDDEN), lambda i: (0, 0)),
            pl.BlockSpec((2 * HIDDEN, 12), lambda i: (0, 0)),
            pl.BlockSpec((1, 12), lambda i: (0, 0)),
            pl.BlockSpec((HIDDEN, HIDDEN), lambda i: (0, 0)),
            pl.BlockSpec((1, HIDDEN), lambda i: (0, 0)),
            pl.BlockSpec((HIDDEN, 3), lambda i: (0, 0)),
            pl.BlockSpec((1, 3), lambda i: (0, 0)),
        ],
        out_specs=[pl.BlockSpec((B, 12), lambda i: (i, 0)),
                   pl.BlockSpec((B, 3), lambda i: (i, 0))],
        out_shape=[jax.ShapeDtypeStruct((N_NODES, 12), jnp.float32),
                   jax.ShapeDtypeStruct((N_NODES, 3), jnp.float32)],
    )(h, p["node_pred1"]["w"], p["node_pred1"]["b"][None, :],
      p["node_pred2"]["w"], p["node_pred2"]["b"][None, :],
      p["pos_pred1"]["w"], p["pos_pred1"]["b"][None, :],
      p["pos_pred2"]["w"], p["pos_pred2"]["b"][None, :])


def kernel(x, edge_index, edge_attr, pos, t, topo_cond, stab_cond, sust_cond,
           batch, params):
    p = params
    scale = DIM_HEAD ** (-0.5)
    half = TIME_DIM // 2
    freqs = jnp.asarray(
        np.exp(np.arange(half, dtype=np.float32) * -(math.log(10000.0) / (half - 1)))
    )[None, :]

    wq_scaled = p["attn_q"] * scale
    wc_blocks, bc_blocks = [], []
    lws = []
    for l in range(NUM_LAYERS):
        gp = p["gnn"][l]
        w1 = gp["msg1"]["w"]
        ws_l, wd_l, we_l = w1[0:64], w1[64:128], w1[128:192]
        wc_blocks.append(p["edge_emb"]["w"] @ we_l)
        bc_blocks.append((p["edge_emb"]["b"] @ we_l + gp["msg1"]["b"])[None, :])
        wu = gp["upd"]["w"]
        lws.append({
            "wsd": jnp.concatenate([ws_l, wd_l], axis=1),
            "wh": wu[0:64],
            "wagg": gp["msg2"]["w"] @ wu[64:128],
            "wcnt": (gp["msg2"]["b"] @ wu[64:128])[None, :],
            "bvec": gp["upd"]["b"][None, :],
            "g": p["ln"][l]["g"][None, :],
            "b": p["ln"][l]["b"][None, :],
        })
    wc_all = jnp.concatenate(wc_blocks, axis=1)
    bc_all = jnp.concatenate(bc_blocks, axis=1)

    silu_te, k8, v8 = _cond_call(t[:, None], topo_cond, stab_cond, sust_cond,
                                 freqs, p)

    kbd = jnp.zeros((INNER, 32), jnp.float32)
    vbd = jnp.zeros((32, INNER), jnp.float32)
    for hh in range(HEADS):
        kbd = kbd.at[32 * hh:32 * (hh + 1), 8 * hh:8 * (hh + 1)].set(
            k8[:, 32 * hh:32 * (hh + 1)].T)
        vbd = vbd.at[8 * hh:8 * (hh + 1), 32 * hh:32 * (hh + 1)].set(
            v8[:, 32 * hh:32 * (hh + 1)])
    wqk = wq_scaled @ kbd
    for l in range(NUM_LAYERS):
        lws[l]["wqk"] = wqk
        lws[l]["vbd"] = vbd
        lws[l]["wao"] = p["attn_out"]["w"]
        lws[l]["bao"] = p["attn_out"]["b"][None, :]

    eps = _edge_const_call(edge_attr, wc_all, bc_all)

    h, ti, hsd = _init_call(
        x, batch[:, None], silu_te,
        p["node_emb"]["w"], p["node_emb"]["b"][None, :], lws[0]["wsd"])

    src3d = edge_index[0].reshape(_NW, _NCHUNK, 1, _CHUNK)
    dst3d = edge_index[1].reshape(_NW, _NCHUNK, 1, _CHUNK)

    for l in range(NUM_LAYERS):
        ep3 = eps[l].reshape(_NW * _NCHUNK, _CHUNK, HIDDEN)
        s2 = _sc_edge_call(hsd, ep3, src3d, dst3d)
        if l < NUM_LAYERS - 1:
            lw = dict(lws[l])
            lw["wsd"] = lws[l + 1]["wsd"]
            h, hsd = _node_call(h, s2, ti, lw, proj=True)
        else:
            h = _node_call(h, s2, ti, lws[l], proj=False)

    node_pred, pos_pred = _heads_call(h, p)
    return node_pred, pos_pred


# speedup vs baseline: 3.5300x; 1.0601x over previous
import functools
import math

import jax
import jax.numpy as jnp
import numpy as np
from jax import lax
from jax.experimental import pallas as pl
from jax.experimental.pallas import tpu as pltpu
from jax.experimental.pallas import tpu_sc as plsc

N_NODES = 10000
N_EDGES = 160000
HIDDEN = 64
NUM_LAYERS = 6
HEADS = 4
DIM_HEAD = 32
INNER = HEADS * DIM_HEAD
N_GRAPHS = 8
TIME_DIM = 64

_NW = 32
_EPW = N_EDGES // _NW
_CHUNK = 40
_NCHUNK = _EPW // _CHUNK
_STRIPE = N_NODES // 16


def _sc_edge_call(tbl, ep3, src3d, dst3d):
    mesh = plsc.VectorSubcoreMesh(core_axis_name="c", subcore_axis_name="s")

    @functools.partial(
        pl.kernel, mesh=mesh,
        out_type=jax.ShapeDtypeStruct((2, 16, _STRIPE, 2 * HIDDEN), jnp.float32),
        scratch_types=[
            pltpu.VMEM((4, 1, _CHUNK), jnp.int32),
            pltpu.VMEM((4, 1, _CHUNK), jnp.int32),
            pltpu.VMEM((2, _CHUNK, 2 * HIDDEN), jnp.float32),
            pltpu.VMEM((2, _CHUNK, 2 * HIDDEN), jnp.float32),
            pltpu.VMEM((2, _CHUNK, HIDDEN), jnp.float32),
            pltpu.VMEM_SHARED((N_NODES, 2 * HIDDEN), jnp.float32),
            pltpu.SemaphoreType.DMA,
            pltpu.SemaphoreType.DMA,
            pltpu.SemaphoreType.DMA,
            pltpu.SemaphoreType.DMA,
            pltpu.SemaphoreType.DMA,
            pltpu.SemaphoreType.DMA,
        ],
    )
    def k(tbl_hbm, ep_hbm, src_hbm, dst_hbm, out_hbm,
          ixs, ixd, rows_s, rows_d, rows_e, s_sh,
          six0, six1, sg0, sg1, ssc0, ssc1):
        c = lax.axis_index("c")
        s = lax.axis_index("s")
        w = c * 16 + s
        ep0 = w * _NCHUNK

        def issue_idx(i, slot):
            sem = [six0, six1][slot % 2]
            return (pltpu.async_copy(src_hbm.at[w, i], ixs.at[slot], sem),
                    pltpu.async_copy(dst_hbm.at[w, i], ixd.at[slot], sem))

        def issue_gathers(i, b, slot):
            sem = [sg0, sg1][b]
            return (pltpu.async_copy(tbl_hbm.at[ixs.at[slot, 0]],
                                     rows_s.at[b], sem),
                    pltpu.async_copy(tbl_hbm.at[ixd.at[slot, 0]],
                                     rows_d.at[b], sem),
                    pltpu.async_copy(ep_hbm.at[ep0 + i], rows_e.at[b], sem))

        def compute(b):
            def body(r0, carry2):
                for r in (2 * r0, 2 * r0 + 1):
                    zs = []
                    for kk in range(HIDDEN // 16):
                        zs.append(rows_s[b, r, pl.ds(16 * kk, 16)]
                                  + rows_d[b, r, pl.ds(HIDDEN + 16 * kk, 16)]
                                  + rows_e[b, r, pl.ds(16 * kk, 16)])
                    for kk in range(HIDDEN // 16):
                        z = zs[kk]
                        rows_d[b, r, pl.ds(16 * kk, 16)] = z / (1.0 + jnp.exp(-z))
                        rows_d[b, r, pl.ds(HIDDEN + 16 * kk, 16)] = jnp.ones(
                            (16,), jnp.float32)
                return carry2
            lax.fori_loop(0, _CHUNK // 2, body, 0)

        def issue_scatter(b, slot):
            return pltpu.async_copy(rows_d.at[b], s_sh.at[ixd.at[slot, 0]],
                                    [ssc0, ssc1][b], add=True)

        def fill(r, carry):
            for kk in range(2 * HIDDEN // 16):
                rows_s[0, r, pl.ds(16 * kk, 16)] = jnp.zeros((16,), jnp.float32)
            return carry
        lax.fori_loop(0, _CHUNK, fill, 0)
        base = s * _STRIPE
        for off in range(0, 600, _CHUNK):
            pltpu.sync_copy(rows_s.at[0], s_sh.at[pl.ds(base + off, _CHUNK)])
        pltpu.sync_copy(rows_s.at[0].at[pl.ds(0, 25)],
                        s_sh.at[pl.ds(base + 600, 25)])
        plsc.subcore_barrier()

        def quad(j, carry):
            a = 4 * j
            i0 = issue_idx(a, 0)
            i1 = issue_idx(a + 1, 1)
            i2 = issue_idx(a + 2, 2)
            i3 = issue_idx(a + 3, 3)
            i0[0].wait()
            i0[1].wait()
            g0 = issue_gathers(a, 0, 0)
            i1[0].wait()
            i1[1].wait()
            g1 = issue_gathers(a + 1, 1, 1)
            for h in g0:
                h.wait()
            compute(0)
            s0 = issue_scatter(0, 0)
            for h in g1:
                h.wait()
            compute(1)
            s1 = issue_scatter(1, 1)
            s0.wait()
            i2[0].wait()
            i2[1].wait()
            g2 = issue_gathers(a + 2, 0, 2)
            s1.wait()
            i3[0].wait()
            i3[1].wait()
            g3 = issue_gathers(a + 3, 1, 3)
            for h in g2:
                h.wait()
            compute(0)
            s2 = issue_scatter(0, 2)
            for h in g3:
                h.wait()
            compute(1)
            s3 = issue_scatter(1, 3)
            s2.wait()
            s3.wait()
            return carry
        lax.fori_loop(0, _NCHUNK // 4, quad, 0)

        ia1, ia2 = issue_idx(_NCHUNK - 1, 0)
        ia1.wait()
        ia2.wait()
        ga = issue_gathers(_NCHUNK - 1, 0, 0)
        for h in ga:
            h.wait()
        compute(0)
        issue_scatter(0, 0).wait()

        plsc.subcore_barrier()
        pltpu.sync_copy(s_sh.at[pl.ds(base, _STRIPE)], out_hbm.at[c, s])

    return k(tbl, ep3, src3d, dst3d).reshape(2, N_NODES, 2 * HIDDEN)


def _cond_body(t_ref, topo_ref, stab_ref, sust_ref, freqs_ref,
               tl1w, tl1b, tl2w, tl2b,
               pl1w, pl1b, pl2w, pl2b,
               sl1w, sl1b, sl2w, sl2b,
               ul1w, ul1b, ul2w, ul2b,
               cl1w, cl1b, cl2w, cl2b,
               akw, avw,
               silu_te_ref, k8_ref, v8_ref):
    silu = jax.nn.silu
    te = t_ref[...] * freqs_ref[...]
    te = jnp.concatenate([jnp.sin(te), jnp.cos(te)], axis=-1)
    te = silu(jnp.dot(te, tl1w[...], preferred_element_type=jnp.float32) + tl1b[...])
    te = jnp.dot(te, tl2w[...], preferred_element_type=jnp.float32) + tl2b[...]
    silu_te_ref[...] = silu(te)
    topo = silu(jnp.dot(topo_ref[...], pl1w[...], preferred_element_type=jnp.float32) + pl1b[...])
    topo = jnp.dot(topo, pl2w[...], preferred_element_type=jnp.float32) + pl2b[...]
    stab = silu(jnp.dot(stab_ref[...], sl1w[...], preferred_element_type=jnp.float32) + sl1b[...])
    stab = jnp.dot(stab, sl2w[...], preferred_element_type=jnp.float32) + sl2b[...]
    sust = silu(jnp.dot(sust_ref[...], ul1w[...], preferred_element_type=jnp.float32) + ul1b[...])
    sust = jnp.dot(sust, ul2w[...], preferred_element_type=jnp.float32) + ul2b[...]
    cc = jnp.concatenate([topo, stab, sust], axis=-1)
    cc = silu(jnp.dot(cc, cl1w[...], preferred_element_type=jnp.float32) + cl1b[...])
    cond = jnp.dot(cc, cl2w[...], preferred_element_type=jnp.float32) + cl2b[...]
    k8_ref[...] = jnp.dot(cond, akw[...], preferred_element_type=jnp.float32)
    v8_ref[...] = jnp.dot(cond, avw[...], preferred_element_type=jnp.float32)


def _cond_call(t2d, topo, stab, sust, freqs, p):
    outs = [
        jax.ShapeDtypeStruct((N_GRAPHS, HIDDEN), jnp.float32),
        jax.ShapeDtypeStruct((N_GRAPHS, INNER), jnp.float32),
        jax.ShapeDtypeStruct((N_GRAPHS, INNER), jnp.float32),
    ]
    args = [t2d, topo, stab, sust, freqs]
    for name in ("time_l1", "time_l2", "topo_l1", "topo_l2", "stab_l1",
                 "stab_l2", "sust_l1", "sust_l2", "comb_l1", "comb_l2"):
        args.append(p[name]["w"])
        args.append(p[name]["b"][None, :])
    args.append(p["attn_k"])
    args.append(p["attn_v"])
    return pl.pallas_call(_cond_body, out_shape=outs)(*args)


def _edge_const_body(ea_ref, wc_ref, bc_ref, *outs):
    r = jnp.dot(ea_ref[...], wc_ref[...], preferred_element_type=jnp.float32) + bc_ref[...]
    for l in range(NUM_LAYERS):
        outs[l][...] = r[:, 64 * l:64 * (l + 1)]


def _edge_const_call(edge_attr, wc_all, bc_all):
    B = 2000
    n = edge_attr.shape[0]
    G = n // B
    return pl.pallas_call(
        _edge_const_body,
        grid=(G,),
        in_specs=[
            pl.BlockSpec((B, 20), lambda i: (i, 0)),
            pl.BlockSpec((20, 64 * NUM_LAYERS), lambda i: (0, 0)),
            pl.BlockSpec((1, 64 * NUM_LAYERS), lambda i: (0, 0)),
        ],
        out_specs=[pl.BlockSpec((B, HIDDEN), lambda i: (i, 0))] * NUM_LAYERS,
        out_shape=[jax.ShapeDtypeStruct((n, HIDDEN), jnp.float32)] * NUM_LAYERS,
    )(edge_attr, wc_all, bc_all)


def _init_body(x_ref, b_ref, te_ref, wemb_ref, bemb_ref, wsd_ref,
               h_ref, ti_ref, hsd_ref):
    h = jnp.dot(x_ref[...], wemb_ref[...], preferred_element_type=jnp.float32) + bemb_ref[...]
    h_ref[...] = h
    oh = (b_ref[...] == lax.broadcasted_iota(jnp.int32, (1, N_GRAPHS), 1)
          ).astype(jnp.float32)
    ti_ref[...] = jnp.dot(oh, te_ref[...], preferred_element_type=jnp.float32)
    hsd_ref[...] = jnp.dot(h, wsd_ref[...], preferred_element_type=jnp.float32)


def _init_call(x, batch2d, silu_te, wemb, bemb, wsd0):
    B = 2000
    G = N_NODES // B
    return pl.pallas_call(
        _init_body,
        grid=(G,),
        in_specs=[
            pl.BlockSpec((B, 12), lambda i: (i, 0)),
            pl.BlockSpec((B, 1), lambda i: (i, 0)),
            pl.BlockSpec((N_GRAPHS, HIDDEN), lambda i: (0, 0)),
            pl.BlockSpec((12, HIDDEN), lambda i: (0, 0)),
            pl.BlockSpec((1, HIDDEN), lambda i: (0, 0)),
            pl.BlockSpec((HIDDEN, 2 * HIDDEN), lambda i: (0, 0)),
        ],
        out_specs=[pl.BlockSpec((B, HIDDEN), lambda i: (i, 0)),
                   pl.BlockSpec((B, HIDDEN), lambda i: (i, 0)),
                   pl.BlockSpec((B, 2 * HIDDEN), lambda i: (i, 0))],
        out_shape=[jax.ShapeDtypeStruct((N_NODES, HIDDEN), jnp.float32),
                   jax.ShapeDtypeStruct((N_NODES, HIDDEN), jnp.float32),
                   jax.ShapeDtypeStruct((N_NODES, 2 * HIDDEN), jnp.float32)],
    )(x, batch2d, silu_te, wemb, bemb, wsd0)


def _attention(hu, kbd_ref, vbd_ref):
    q = jnp.dot(hu, kbd_ref[...], preferred_element_type=jnp.float32)
    ps = []
    for hh in range(HEADS):
        sl = q[:, 8 * hh:8 * (hh + 1)]
        m = jnp.max(sl, axis=-1, keepdims=True)
        e = jnp.exp(sl - m)
        ps.append(e / jnp.sum(e, axis=-1, keepdims=True))
    P = jnp.concatenate(ps, axis=-1)
    return jnp.dot(P, vbd_ref[...], preferred_element_type=jnp.float32)


def _node_update(h_ref, s2_ref, ti_ref, wh_ref, wagg_ref, bvec_ref,
                 wcnt_ref, g_ref, b_ref, wqk_ref, vbd_ref, wao_ref, bao_ref):
    h = h_ref[...]
    acc = s2_ref[0] + s2_ref[1]
    S = acc[:, 0:HIDDEN]
    cnt = acc[:, HIDDEN:HIDDEN + 1]
    hu = (jnp.dot(h, wh_ref[...], preferred_element_type=jnp.float32)
          + jnp.dot(S, wagg_ref[...], preferred_element_type=jnp.float32)
          + cnt * wcnt_ref[...] + bvec_ref[...])
    mu = jnp.mean(hu, axis=-1, keepdims=True)
    var = jnp.mean((hu - mu) * (hu - mu), axis=-1, keepdims=True)
    hu = (hu - mu) * lax.rsqrt(var + 1e-5) * g_ref[...] + b_ref[...]
    att = _attention(hu, wqk_ref, vbd_ref)
    lin = jnp.dot(att, wao_ref[...], preferred_element_type=jnp.float32) + bao_ref[...]
    return h + lin + ti_ref[...]


def _node_body_proj(h_ref, s2_ref, ti_ref, wh_ref, wagg_ref,
                    bvec_ref, wcnt_ref, g_ref, b_ref, wqk_ref, vbd_ref,
                    wao_ref, bao_ref, wsd_ref,
                    hn_ref, hsd_ref):
    hn = _node_update(h_ref, s2_ref, ti_ref, wh_ref, wagg_ref,
                      bvec_ref, wcnt_ref, g_ref, b_ref, wqk_ref, vbd_ref,
                      wao_ref, bao_ref)
    hn_ref[...] = hn
    hsd_ref[...] = jnp.dot(hn, wsd_ref[...], preferred_element_type=jnp.float32)


def _node_body_last(h_ref, s2_ref, ti_ref, wh_ref, wagg_ref,
                    bvec_ref, wcnt_ref, g_ref, b_ref, wqk_ref, vbd_ref,
                    wao_ref, bao_ref,
                    n1w, n1b, n2w, n2b, p1w, p1b, p2w, p2b,
                    np_ref, pp_ref):
    hn = _node_update(h_ref, s2_ref, ti_ref, wh_ref,
                      wagg_ref, bvec_ref, wcnt_ref, g_ref, b_ref,
                      wqk_ref, vbd_ref, wao_ref, bao_ref)
    silu = jax.nn.silu
    a = silu(jnp.dot(hn, n1w[...], preferred_element_type=jnp.float32) + n1b[...])
    np_ref[...] = jnp.dot(a, n2w[...], preferred_element_type=jnp.float32) + n2b[...]
    bb = silu(jnp.dot(hn, p1w[...], preferred_element_type=jnp.float32) + p1b[...])
    pp_ref[...] = jnp.dot(bb, p2w[...], preferred_element_type=jnp.float32) + p2b[...]


def _node_call(h, s2, ti, lw, proj, heads_w=None):
    B = 2000
    G = N_NODES // B
    common_in = [
        pl.BlockSpec((B, HIDDEN), lambda i: (i, 0)),
        pl.BlockSpec((2, B, 2 * HIDDEN), lambda i: (0, i, 0)),
        pl.BlockSpec((B, HIDDEN), lambda i: (i, 0)),
        pl.BlockSpec((HIDDEN, HIDDEN), lambda i: (0, 0)),
        pl.BlockSpec((HIDDEN, HIDDEN), lambda i: (0, 0)),
        pl.BlockSpec((1, HIDDEN), lambda i: (0, 0)),
        pl.BlockSpec((1, HIDDEN), lambda i: (0, 0)),
        pl.BlockSpec((1, HIDDEN), lambda i: (0, 0)),
        pl.BlockSpec((1, HIDDEN), lambda i: (0, 0)),
        pl.BlockSpec((HIDDEN, 32), lambda i: (0, 0)),
        pl.BlockSpec((32, INNER), lambda i: (0, 0)),
        pl.BlockSpec((INNER, HIDDEN), lambda i: (0, 0)),
        pl.BlockSpec((1, HIDDEN), lambda i: (0, 0)),
    ]
    args = [h, s2, ti, lw["wh"], lw["wagg"], lw["bvec"], lw["wcnt"],
            lw["g"], lw["b"], lw["wqk"], lw["vbd"], lw["wao"], lw["bao"]]
    if proj:
        in_specs = common_in + [
            pl.BlockSpec((HIDDEN, 2 * HIDDEN), lambda i: (0, 0)),
        ]
        return pl.pallas_call(
            _node_body_proj,
            grid=(G,),
            in_specs=in_specs,
            out_specs=[pl.BlockSpec((B, HIDDEN), lambda i: (i, 0)),
                       pl.BlockSpec((B, 2 * HIDDEN), lambda i: (i, 0))],
            out_shape=[jax.ShapeDtypeStruct((N_NODES, HIDDEN), jnp.float32),
                       jax.ShapeDtypeStruct((N_NODES, 2 * HIDDEN), jnp.float32)],
        )(*args, lw["wsd"])
    in_specs = common_in + [
        pl.BlockSpec((HIDDEN, 2 * HIDDEN), lambda i: (0, 0)),
        pl.BlockSpec((1, 2 * HIDDEN), lambda i: (0, 0)),
        pl.BlockSpec((2 * HIDDEN, 12), lambda i: (0, 0)),
        pl.BlockSpec((1, 12), lambda i: (0, 0)),
        pl.BlockSpec((HIDDEN, HIDDEN), lambda i: (0, 0)),
        pl.BlockSpec((1, HIDDEN), lambda i: (0, 0)),
        pl.BlockSpec((HIDDEN, 3), lambda i: (0, 0)),
        pl.BlockSpec((1, 3), lambda i: (0, 0)),
    ]
    return pl.pallas_call(
        _node_body_last,
        grid=(G,),
        in_specs=in_specs,
        out_specs=[pl.BlockSpec((B, 12), lambda i: (i, 0)),
                   pl.BlockSpec((B, 3), lambda i: (i, 0))],
        out_shape=[jax.ShapeDtypeStruct((N_NODES, 12), jnp.float32),
                   jax.ShapeDtypeStruct((N_NODES, 3), jnp.float32)],
    )(*args, *heads_w)


def _heads_body(h_ref, n1w, n1b, n2w, n2b, p1w, p1b, p2w, p2b,
                np_ref, pp_ref):
    silu = jax.nn.silu
    h = h_ref[...]
    a = silu(jnp.dot(h, n1w[...], preferred_element_type=jnp.float32) + n1b[...])
    np_ref[...] = jnp.dot(a, n2w[...], preferred_element_type=jnp.float32) + n2b[...]
    b = silu(jnp.dot(h, p1w[...], preferred_element_type=jnp.float32) + p1b[...])
    pp_ref[...] = jnp.dot(b, p2w[...], preferred_element_type=jnp.float32) + p2b[...]


def _heads_call(h, p):
    B = 1000
    G = N_NODES // B
    return pl.pallas_call(
        _heads_body,
        grid=(G,),
        in_specs=[
            pl.BlockSpec((B, HIDDEN), lambda i: (i, 0)),
            pl.BlockSpec((HIDDEN, 2 * HIDDEN), lambda i: (0, 0)),
            pl.BlockSpec((1, 2 * HIDDEN), lambda i: (0, 0)),
            pl.BlockSpec((2 * HIDDEN, 12), lambda i: (0, 0)),
            pl.BlockSpec((1, 12), lambda i: (0, 0)),
            pl.BlockSpec((HIDDEN, HIDDEN), lambda i: (0, 0)),
            pl.BlockSpec((1, HIDDEN), lambda i: (0, 0)),
            pl.BlockSpec((HIDDEN, 3), lambda i: (0, 0)),
            pl.BlockSpec((1, 3), lambda i: (0, 0)),
        ],
        out_specs=[pl.BlockSpec((B, 12), lambda i: (i, 0)),
                   pl.BlockSpec((B, 3), lambda i: (i, 0))],
        out_shape=[jax.ShapeDtypeStruct((N_NODES, 12), jnp.float32),
                   jax.ShapeDtypeStruct((N_NODES, 3), jnp.float32)],
    )(h, p["node_pred1"]["w"], p["node_pred1"]["b"][None, :],
      p["node_pred2"]["w"], p["node_pred2"]["b"][None, :],
      p["pos_pred1"]["w"], p["pos_pred1"]["b"][None, :],
      p["pos_pred2"]["w"], p["pos_pred2"]["b"][None, :])


def kernel(x, edge_index, edge_attr, pos, t, topo_cond, stab_cond, sust_cond,
           batch, params):
    p = params
    scale = DIM_HEAD ** (-0.5)
    half = TIME_DIM // 2
    freqs = jnp.asarray(
        np.exp(np.arange(half, dtype=np.float32) * -(math.log(10000.0) / (half - 1)))
    )[None, :]

    wq_scaled = p["attn_q"] * scale
    wc_blocks, bc_blocks = [], []
    lws = []
    for l in range(NUM_LAYERS):
        gp = p["gnn"][l]
        w1 = gp["msg1"]["w"]
        ws_l, wd_l, we_l = w1[0:64], w1[64:128], w1[128:192]
        wc_blocks.append(p["edge_emb"]["w"] @ we_l)
        bc_blocks.append((p["edge_emb"]["b"] @ we_l + gp["msg1"]["b"])[None, :])
        wu = gp["upd"]["w"]
        lws.append({
            "wsd": jnp.concatenate([ws_l, wd_l], axis=1),
            "wh": wu[0:64],
            "wagg": gp["msg2"]["w"] @ wu[64:128],
            "wcnt": (gp["msg2"]["b"] @ wu[64:128])[None, :],
            "bvec": gp["upd"]["b"][None, :],
            "g": p["ln"][l]["g"][None, :],
            "b": p["ln"][l]["b"][None, :],
        })
    wc_all = jnp.concatenate(wc_blocks, axis=1)
    bc_all = jnp.concatenate(bc_blocks, axis=1)

    silu_te, k8, v8 = _cond_call(t[:, None], topo_cond, stab_cond, sust_cond,
                                 freqs, p)

    kbd = jnp.zeros((INNER, 32), jnp.float32)
    vbd = jnp.zeros((32, INNER), jnp.float32)
    for hh in range(HEADS):
        kbd = kbd.at[32 * hh:32 * (hh + 1), 8 * hh:8 * (hh + 1)].set(
            k8[:, 32 * hh:32 * (hh + 1)].T)
        vbd = vbd.at[8 * hh:8 * (hh + 1), 32 * hh:32 * (hh + 1)].set(
            v8[:, 32 * hh:32 * (hh + 1)])
    wqk = wq_scaled @ kbd
    for l in range(NUM_LAYERS):
        lws[l]["wqk"] = wqk
        lws[l]["vbd"] = vbd
        lws[l]["wao"] = p["attn_out"]["w"]
        lws[l]["bao"] = p["attn_out"]["b"][None, :]

    eps = _edge_const_call(edge_attr, wc_all, bc_all)

    h, ti, hsd = _init_call(
        x, batch[:, None], silu_te,
        p["node_emb"]["w"], p["node_emb"]["b"][None, :], lws[0]["wsd"])

    src3d = edge_index[0].reshape(_NW, _NCHUNK, 1, _CHUNK)
    dst3d = edge_index[1].reshape(_NW, _NCHUNK, 1, _CHUNK)

    for l in range(NUM_LAYERS):
        ep3 = eps[l].reshape(_NW * _NCHUNK, _CHUNK, HIDDEN)
        s2 = _sc_edge_call(hsd, ep3, src3d, dst3d)
        if l < NUM_LAYERS - 1:
            lw = dict(lws[l])
            lw["wsd"] = lws[l + 1]["wsd"]
            h, hsd = _node_call(h, s2, ti, lw, proj=True)
        else:
            heads_w = [p["node_pred1"]["w"], p["node_pred1"]["b"][None, :],
                       p["node_pred2"]["w"], p["node_pred2"]["b"][None, :],
                       p["pos_pred1"]["w"], p["pos_pred1"]["b"][None, :],
                       p["pos_pred2"]["w"], p["pos_pred2"]["b"][None, :]]
            node_pred, pos_pred = _node_call(h, s2, ti, lws[l], proj=False,
                                             heads_w=heads_w)
    return node_pred, pos_pred
